# baseline (speedup 1.0000x reference)
import functools

import jax
import jax.numpy as jnp
from jax import lax
from jax.experimental import pallas as pl
from jax.experimental.pallas import tpu as pltpu
from jax.experimental.pallas import tpu_sc as plsc

VOCAB = 1000000
EMBED_DIM = 64
BATCH = 16384
HIST = 50

NUM_CORES = 2
NUM_SUBCORES = 16
NW = NUM_CORES * NUM_SUBCORES
B_PER_W = BATCH // NW
LB = 128
BHW = B_PER_W // LB
NSLAB = BHW * HIST
NBUF = 4


def _gather_kernel(table_hbm, idx_hbm, out_hbm, idx_v, idxt_v, rows_v, slab_v,
                   gsem0, gsem1, gsem2, gsem3, osem0, osem1, osem2, osem3):
    gsems = (gsem0, gsem1, gsem2, gsem3)
    osems = (osem0, osem1, osem2, osem3)
    wid = lax.axis_index("s") * NUM_CORES + lax.axis_index("c")
    b0 = wid * B_PER_W
    bh0 = wid * BHW

    pltpu.sync_copy(idx_hbm.at[pl.ds(b0, B_PER_W)], idx_v)

    lanes = lax.iota(jnp.int32, 16)
    row_vecs = [g * 16 + lanes for g in range(LB // 16)]

    def tbody(t, carry):
        for g in range(B_PER_W // 16):
            vals = plsc.load_gather(idx_v, [row_vecs[g % (LB // 16)] +
                                            (g // (LB // 16)) * LB,
                                            jnp.full((16,), t, jnp.int32)])
            idxt_v[t, pl.ds(g * 16, 16)] = vals
        return carry

    lax.fori_loop(0, HIST, tbody, 0)

    def start_gather(s, buf):
        bh_l = s // HIST
        t = s % HIST
        pltpu.async_copy(
            table_hbm.at[idxt_v.at[t, pl.ds(bh_l * LB, LB)]],
            rows_v.at[buf], gsems[buf])

    def wait_gather(buf):
        pltpu.make_async_copy(table_hbm.at[idxt_v.at[0, pl.ds(0, LB)]],
                              rows_v.at[buf], gsems[buf]).wait()

    def start_out(s, buf):
        bh_l = s // HIST
        t = s % HIST
        pltpu.async_copy(slab_v.at[buf],
                         out_hbm.at[t, :, bh0 + bh_l], osems[buf])

    def wait_out(buf):
        pltpu.make_async_copy(slab_v.at[buf],
                              out_hbm.at[0, :, 0], osems[buf]).wait()

    def assemble(buf):
        rows = rows_v.at[buf]

        def abody(k, carry):
            ch = k // 8
            cl = lax.rem(k, 8)
            col = jnp.full((16,), k, jnp.int32)
            for blg in range(LB // 16):
                vals = plsc.load_gather(rows, [row_vecs[blg], col])
                slab_v[buf, ch, cl, pl.ds(blg * 16, 16)] = vals
            return carry

        lax.fori_loop(0, EMBED_DIM, abody, 0)

    for n in range(NBUF):
        start_gather(n, n)

    def body(g, carry):
        for n in range(NBUF):
            s = g * NBUF + n
            wait_gather(n)

            @pl.when(s >= NBUF)
            def _():
                wait_out(n)

            assemble(n)
            start_out(s, n)

            @pl.when(s + NBUF < NSLAB)
            def _():
                start_gather(s + NBUF, n)

        return carry

    lax.fori_loop(0, NSLAB // NBUF, body, 0)

    for n in range(NBUF):
        wait_out(n)


@jax.jit
def _lookup(nodes_ids, table):
    mesh = plsc.VectorSubcoreMesh(core_axis_name="c", subcore_axis_name="s")
    out5d = pl.kernel(
        _gather_kernel,
        out_type=jax.ShapeDtypeStruct(
            (HIST, EMBED_DIM // 8, BATCH // LB, 8, LB), jnp.float32),
        mesh=mesh,
        scratch_types=[
            pltpu.VMEM((B_PER_W, HIST), jnp.int32),
            pltpu.VMEM((HIST, B_PER_W), jnp.int32),
            pltpu.VMEM((NBUF, LB, EMBED_DIM), jnp.float32),
            pltpu.VMEM((NBUF, EMBED_DIM // 8, 8, LB), jnp.float32),
        ] + [pltpu.SemaphoreType.DMA] * (2 * NBUF),
        compiler_params=pltpu.CompilerParams(use_tc_tiling_on_sc=False,
                                             needs_layout_passes=False),
    )(table, nodes_ids)
    return out5d.transpose(2, 4, 0, 1, 3).reshape(BATCH, HIST, EMBED_DIM)


def kernel(nodes_ids, table):
    return _lookup(nodes_ids, table)


# speedup vs baseline: 1.3451x; 1.3451x over previous
import functools

import jax
import jax.numpy as jnp
from jax import lax
from jax.experimental import pallas as pl
from jax.experimental.pallas import tpu as pltpu
from jax.experimental.pallas import tpu_sc as plsc

VOCAB = 1000000
EMBED_DIM = 64
BATCH = 16384
HIST = 50

NUM_CORES = 2
NUM_SUBCORES = 16
NW = NUM_CORES * NUM_SUBCORES
B_PER_W = BATCH // NW
GB = 8
NCHUNK = B_PER_W // GB
NBUF = 2
NSPLIT = 2
HSUB = HIST // NSPLIT


def _gather_kernel(table_hbm, idx_hbm, out_hbm, idx_v, rows_v,
                   gsem0, gsem1, osem0, osem1):
    gsems = (gsem0, gsem1)
    osems = (osem0, osem1)
    wid = lax.axis_index("s") * NUM_CORES + lax.axis_index("c")
    b0 = wid * B_PER_W

    pltpu.sync_copy(idx_hbm.at[pl.ds(b0, B_PER_W)], idx_v)

    def start_slab(ch, buf):
        for j in range(GB):
            b = ch * GB + j
            pltpu.async_copy(table_hbm.at[idx_v.at[b]],
                             rows_v.at[buf].at[j], gsems[buf])

    def wait_slab(buf):
        for j in range(GB):
            pltpu.make_async_copy(table_hbm.at[idx_v.at[0]],
                                  rows_v.at[buf].at[j], gsems[buf]).wait()

    def start_out(ch, buf):
        pltpu.async_copy(rows_v.at[buf],
                         out_hbm.at[pl.ds(b0 + ch * GB, GB)], osems[buf])

    def wait_out(buf):
        pltpu.make_async_copy(rows_v.at[buf],
                              out_hbm.at[pl.ds(0, GB)], osems[buf]).wait()

    for n in range(NBUF):
        start_slab(n, n)

    def body(g, carry):
        for n in range(NBUF):
            ch = g * NBUF + n
            wait_slab(n)
            start_out(ch, n)

            @pl.when(ch + NBUF < NCHUNK)
            def _():
                wait_out(n)
                start_slab(ch + NBUF, n)

        return carry

    lax.fori_loop(0, NCHUNK // NBUF, body, 0)

    for n in range(NBUF):
        wait_out(n)


def _lookup_chunk(nodes_ids_chunk, table):
    mesh = plsc.VectorSubcoreMesh(core_axis_name="c", subcore_axis_name="s")
    return pl.kernel(
        _gather_kernel,
        out_type=jax.ShapeDtypeStruct((BATCH, HSUB, EMBED_DIM), jnp.float32),
        mesh=mesh,
        scratch_types=[
            pltpu.VMEM((B_PER_W, HSUB), jnp.int32),
            pltpu.VMEM((NBUF, GB, HSUB, EMBED_DIM), jnp.float32),
        ] + [pltpu.SemaphoreType.DMA] * (2 * NBUF),
        compiler_params=pltpu.CompilerParams(use_tc_tiling_on_sc=False),
    )(table, nodes_ids_chunk)


@jax.jit
def _lookup(nodes_ids, table):
    outs = [
        _lookup_chunk(
            lax.slice_in_dim(nodes_ids, k * HSUB, (k + 1) * HSUB, axis=1),
            table)
        for k in range(NSPLIT)
    ]
    return jnp.concatenate(outs, axis=1)


def kernel(nodes_ids, table):
    return _lookup(nodes_ids, table)


# speedup vs baseline: 1.8295x; 1.3601x over previous
import functools

import jax
import jax.numpy as jnp
from jax import lax
from jax.experimental import pallas as pl
from jax.experimental.pallas import tpu as pltpu
from jax.experimental.pallas import tpu_sc as plsc

VOCAB = 1000000
EMBED_DIM = 64
BATCH = 16384
HIST = 50

NUM_CORES = 2
NUM_SUBCORES = 16
NW = NUM_CORES * NUM_SUBCORES
B_PER_W = BATCH // NW
LB = 128
BHW = B_PER_W // LB
NSLAB = BHW * HIST
NBUF = 4
SLAB_MINOR = LB + 1


def _gather_kernel(table_hbm, idx_hbm, out_hbm, idx_v, idxt_v, rows_v, slab_v,
                   gsem0, gsem1, gsem2, gsem3, osem0, osem1, osem2, osem3):
    gsems = (gsem0, gsem1, gsem2, gsem3)
    osems = (osem0, osem1, osem2, osem3)
    wid = lax.axis_index("s") * NUM_CORES + lax.axis_index("c")
    b0 = wid * B_PER_W
    bh0 = wid * BHW

    pltpu.sync_copy(idx_hbm.at[pl.ds(b0, B_PER_W)], idx_v)

    lanes = lax.iota(jnp.int32, 16)
    row_vecs = [g * 16 + lanes for g in range(LB // 16)]
    ch_vecs = [(c0 * 16 + lanes) // 8 for c0 in range(EMBED_DIM // 16)]
    cl_vecs = [lax.rem(c0 * 16 + lanes, 8) for c0 in range(EMBED_DIM // 16)]

    def tbody(t, carry):
        for g in range(B_PER_W // 16):
            vals = plsc.load_gather(idx_v, [row_vecs[g % (LB // 16)] +
                                            (g // (LB // 16)) * LB,
                                            jnp.full((16,), t, jnp.int32)])
            idxt_v[t, pl.ds(g * 16, 16)] = vals
        return carry

    lax.fori_loop(0, HIST, tbody, 0)

    def start_gather(s, buf):
        bh_l = s // HIST
        t = s % HIST
        pltpu.async_copy(
            table_hbm.at[idxt_v.at[t, pl.ds(bh_l * LB, LB)]],
            rows_v.at[buf], gsems[buf])

    def wait_gather(buf):
        pltpu.make_async_copy(table_hbm.at[idxt_v.at[0, pl.ds(0, LB)]],
                              rows_v.at[buf], gsems[buf]).wait()

    def start_out(s, buf):
        bh_l = s // HIST
        t = s % HIST
        pltpu.async_copy(slab_v.at[buf, :, :, pl.ds(0, LB)],
                         out_hbm.at[t, :, bh0 + bh_l], osems[buf])

    def wait_out(buf):
        pltpu.make_async_copy(slab_v.at[buf, :, :, pl.ds(0, LB)],
                              out_hbm.at[0, :, 0], osems[buf]).wait()

    def assemble(buf):
        rows = rows_v.at[buf]
        slab = slab_v.at[buf]

        def abody(bq, carry):
            for i in range(4):
                b = bq * 4 + i
                blv = jnp.full((16,), b, jnp.int32)
                for c0 in range(EMBED_DIM // 16):
                    v = rows[b, pl.ds(c0 * 16, 16)]
                    plsc.store_scatter(slab, [ch_vecs[c0], cl_vecs[c0], blv],
                                       v)
            return carry

        lax.fori_loop(0, LB // 4, abody, 0)

    for n in range(NBUF):
        start_gather(n, n)

    def body(g, carry):
        for n in range(NBUF):
            s = g * NBUF + n
            wait_gather(n)

            @pl.when(s >= NBUF)
            def _():
                wait_out(n)

            assemble(n)
            start_out(s, n)

            @pl.when(s + NBUF < NSLAB)
            def _():
                start_gather(s + NBUF, n)

        return carry

    lax.fori_loop(0, NSLAB // NBUF, body, 0)

    for n in range(NBUF):
        wait_out(n)


@jax.jit
def _lookup(nodes_ids, table):
    mesh = plsc.VectorSubcoreMesh(core_axis_name="c", subcore_axis_name="s")
    out5d = pl.kernel(
        _gather_kernel,
        out_type=jax.ShapeDtypeStruct(
            (HIST, EMBED_DIM // 8, BATCH // LB, 8, LB), jnp.float32),
        mesh=mesh,
        scratch_types=[
            pltpu.VMEM((B_PER_W, HIST), jnp.int32),
            pltpu.VMEM((HIST, B_PER_W), jnp.int32),
            pltpu.VMEM((NBUF, LB, EMBED_DIM), jnp.float32),
            pltpu.VMEM((NBUF, EMBED_DIM // 8, 8, SLAB_MINOR), jnp.float32),
        ] + [pltpu.SemaphoreType.DMA] * (2 * NBUF),
        compiler_params=pltpu.CompilerParams(use_tc_tiling_on_sc=False,
                                             needs_layout_passes=False),
    )(table, nodes_ids)
    return out5d.transpose(2, 4, 0, 1, 3).reshape(BATCH, HIST, EMBED_DIM)


def kernel(nodes_ids, table):
    return _lookup(nodes_ids, table)


# speedup vs baseline: 1.9512x; 1.0665x over previous
import functools

import jax
import jax.numpy as jnp
from jax import lax
from jax.experimental import pallas as pl
from jax.experimental.pallas import tpu as pltpu
from jax.experimental.pallas import tpu_sc as plsc

VOCAB = 1000000
EMBED_DIM = 64
BATCH = 16384
HIST = 50

NUM_CORES = 2
NUM_SUBCORES = 16
NW = NUM_CORES * NUM_SUBCORES
B_PER_W = BATCH // NW
LB = 128
BHW = B_PER_W // LB
NSLAB = BHW * HIST
NBUF = 4
SLAB_MINOR = LB + 1


def _gather_kernel(table_hbm, idx_hbm, out_hbm, idx_v, idxt_v, rows_v, slab_v,
                   gsem0, gsem1, gsem2, gsem3, osem0, osem1, osem2, osem3):
    gsems = (gsem0, gsem1, gsem2, gsem3)
    osems = (osem0, osem1, osem2, osem3)
    wid = lax.axis_index("s") * NUM_CORES + lax.axis_index("c")
    b0 = wid * B_PER_W
    bh0 = wid * BHW

    pltpu.sync_copy(idx_hbm.at[pl.ds(b0, B_PER_W)], idx_v)

    lanes = lax.iota(jnp.int32, 16)
    row_vecs = [g * 16 + lanes for g in range(LB // 16)]
    ch_vecs = [(c0 * 16 + lanes) // 8 for c0 in range(EMBED_DIM // 16)]
    cl_vecs = [lax.rem(c0 * 16 + lanes, 8) for c0 in range(EMBED_DIM // 16)]

    def tbody(t, carry):
        for g in range(B_PER_W // 16):
            vals = plsc.load_gather(idx_v, [row_vecs[g % (LB // 16)] +
                                            (g // (LB // 16)) * LB,
                                            jnp.full((16,), t, jnp.int32)])
            idxt_v[t, pl.ds(g * 16, 16)] = vals + vals
        return carry

    lax.fori_loop(0, HIST, tbody, 0)

    def start_gather(s, buf):
        bh_l = s // HIST
        t = s % HIST
        pltpu.async_copy(
            table_hbm.at[idxt_v.at[t, pl.ds(bh_l * LB, LB)]],
            rows_v.at[buf], gsems[buf])

    def wait_gather(buf):
        pltpu.make_async_copy(table_hbm.at[idxt_v.at[0, pl.ds(0, LB)]],
                              rows_v.at[buf], gsems[buf]).wait()

    def start_out(s, buf):
        bh_l = s // HIST
        t = s % HIST
        pltpu.async_copy(slab_v.at[buf, :, :, pl.ds(0, LB)],
                         out_hbm.at[t, :, bh0 + bh_l], osems[buf])

    def wait_out(buf):
        pltpu.make_async_copy(slab_v.at[buf, :, :, pl.ds(0, LB)],
                              out_hbm.at[0, :, 0], osems[buf]).wait()

    def assemble(buf):
        rows = rows_v.at[buf]
        slab = slab_v.at[buf]

        def abody(bq, carry):
            for i in range(4):
                b = bq * 4 + i
                blv = jnp.full((16,), b, jnp.int32)
                for c0 in range(EMBED_DIM // 16):
                    v = rows[b, pl.ds(c0 * 16, 16)]
                    plsc.store_scatter(slab, [ch_vecs[c0], cl_vecs[c0], blv],
                                       v)
            return carry

        lax.fori_loop(0, LB // 4, abody, 0)

    for n in range(NBUF):
        start_gather(n, n)

    def body(g, carry):
        for n in range(NBUF):
            s = g * NBUF + n
            wait_gather(n)

            @pl.when(s >= NBUF)
            def _():
                wait_out(n)

            assemble(n)
            start_out(s, n)

            @pl.when(s + NBUF < NSLAB)
            def _():
                start_gather(s + NBUF, n)

        return carry

    lax.fori_loop(0, NSLAB // NBUF, body, 0)

    for n in range(NBUF):
        wait_out(n)


@jax.jit
def _lookup(nodes_ids, table):
    table2 = jnp.pad(table, ((0, 0), (0, 128 - EMBED_DIM))).reshape(
        2 * VOCAB, EMBED_DIM)
    mesh = plsc.VectorSubcoreMesh(core_axis_name="c", subcore_axis_name="s")
    out5d = pl.kernel(
        _gather_kernel,
        out_type=jax.ShapeDtypeStruct(
            (HIST, EMBED_DIM // 8, BATCH // LB, 8, LB), jnp.float32),
        mesh=mesh,
        scratch_types=[
            pltpu.VMEM((B_PER_W, HIST), jnp.int32),
            pltpu.VMEM((HIST, B_PER_W), jnp.int32),
            pltpu.VMEM((NBUF, LB, EMBED_DIM), jnp.float32),
            pltpu.VMEM((NBUF, EMBED_DIM // 8, 8, SLAB_MINOR), jnp.float32),
        ] + [pltpu.SemaphoreType.DMA] * (2 * NBUF),
        compiler_params=pltpu.CompilerParams(use_tc_tiling_on_sc=False,
                                             needs_layout_passes=False),
    )(table2, nodes_ids)
    return out5d.transpose(2, 4, 0, 1, 3).reshape(BATCH, HIST, EMBED_DIM)


def kernel(nodes_ids, table):
    return _lookup(nodes_ids, table)
